# baseline (speedup 1.0000x reference)
import functools

import jax
import jax.numpy as jnp
from jax import lax
from jax.experimental import pallas as pl
from jax.experimental.pallas import tpu as pltpu
from jax.experimental.pallas import tpu_sc as plsc

N = 100000
D = 128
V = 95
NW = 32
B_MAIN = 3128
B_LAST = N - 31 * B_MAIN
BLK = 128
NBUF = 7
LOOK = 6
NF_MAIN = B_MAIN // BLK
NF_LAST = B_LAST // BLK
T_MAIN = B_MAIN - NF_MAIN * BLK
T_LAST = B_LAST - NF_LAST * BLK


def _make_kernel():
    mesh = plsc.VectorSubcoreMesh(core_axis_name="c", subcore_axis_name="s")

    @functools.partial(
        pl.kernel,
        mesh=mesh,
        out_type=jax.ShapeDtypeStruct((N, D), jnp.float32),
        scratch_types=[
            pltpu.VMEM_SHARED((V, D), jnp.float32),
            pltpu.VMEM((B_MAIN,), jnp.int32),
            pltpu.VMEM((NBUF, BLK, D), jnp.float32),
            pltpu.SemaphoreType.DMA((NBUF,)),
            pltpu.SemaphoreType.DMA((NBUF,)),
            pltpu.SemaphoreType.DMA,
        ],
    )
    def k(table_hbm, idx_hbm, out_hbm, table_sh, idx_v, rows, sem_g, sem_w,
          sem_i):
        cid = lax.axis_index("c")
        sid = lax.axis_index("s")
        wid = sid * 2 + cid
        last = wid == NW - 1
        base = wid * B_MAIN
        nfull = jnp.where(last, NF_LAST, NF_MAIN)

        @pl.when(jnp.logical_not(last))
        def _():
            pltpu.make_async_copy(
                idx_hbm.at[pl.ds(base, B_MAIN)], idx_v, sem_i).start()

        @pl.when(last)
        def _():
            pltpu.make_async_copy(
                idx_hbm.at[pl.ds(31 * B_MAIN, B_LAST)],
                idx_v.at[pl.ds(0, B_LAST)], sem_i).start()

        @pl.when(jnp.logical_not(last))
        def _():
            pltpu.make_async_copy(
                idx_hbm.at[pl.ds(base, B_MAIN)], idx_v, sem_i).wait()

        @pl.when(last)
        def _():
            pltpu.make_async_copy(
                idx_hbm.at[pl.ds(31 * B_MAIN, B_LAST)],
                idx_v.at[pl.ds(0, B_LAST)], sem_i).wait()

        def g_copy(j, b):
            return pltpu.make_async_copy(
                table_hbm.at[idx_v.at[pl.ds(j * BLK, BLK)]],
                rows.at[b],
                sem_g.at[b],
            )

        def w_copy(j, b):
            return pltpu.make_async_copy(
                rows.at[b],
                out_hbm.at[pl.ds(base + j * BLK, BLK)],
                sem_w.at[b],
            )

        for kk in range(LOOK):
            g_copy(kk, kk).start()

        def body(j, carry):
            b = j % NBUF
            g_copy(j, b).wait()
            w_copy(j, b).start()
            nxt = j + LOOK

            @pl.when(nxt < nfull)
            def _():
                @pl.when(j >= 1)
                def _():
                    w_copy(j - 1, (j - 1) % NBUF).wait()

                g_copy(nxt, nxt % NBUF).start()

            return carry

        lax.fori_loop(0, nfull, body, 0)

        w_copy(nfull - NBUF, (nfull - NBUF) % NBUF).wait()

        def tail(toff, tsz):
            b = (toff // BLK) % NBUF
            pltpu.make_async_copy(
                table_hbm.at[idx_v.at[pl.ds(toff, tsz)]],
                rows.at[b, pl.ds(0, tsz)],
                sem_g.at[b],
            ).start()
            pltpu.make_async_copy(
                table_hbm.at[idx_v.at[pl.ds(toff, tsz)]],
                rows.at[b, pl.ds(0, tsz)],
                sem_g.at[b],
            ).wait()
            pltpu.make_async_copy(
                rows.at[b, pl.ds(0, tsz)],
                out_hbm.at[pl.ds(base + toff, tsz)],
                sem_w.at[b],
            ).start()
            pltpu.make_async_copy(
                rows.at[b, pl.ds(0, tsz)],
                out_hbm.at[pl.ds(base + toff, tsz)],
                sem_w.at[b],
            ).wait()

        @pl.when(jnp.logical_not(last))
        def _():
            tail(NF_MAIN * BLK, T_MAIN)

        @pl.when(last)
        def _():
            tail(NF_LAST * BLK, T_LAST)

        def drain(j, carry):
            w_copy(j, j % NBUF).wait()
            return carry

        lax.fori_loop(nfull - LOOK, nfull, drain, 0)

    return k


_kernel = _make_kernel()


def kernel(atomic_num, embedding_table):
    idx = atomic_num.astype(jnp.int32)
    return _kernel(embedding_table, idx)


# speedup vs baseline: 4.0800x; 4.0800x over previous
import functools

import jax
import jax.numpy as jnp
from jax import lax
from jax.experimental import pallas as pl
from jax.experimental.pallas import tpu as pltpu
from jax.experimental.pallas import tpu_sc as plsc

N = 100000
D = 128
V = 95
NW = 32
B_MAIN = 3128
B_LAST = N - 31 * B_MAIN
BLK = 112
NBUF = 8
LOOK = 5
NFULL = 27
T_MAIN = B_MAIN - NFULL * BLK
T_LAST = B_LAST - NFULL * BLK


def _make_kernel():
    mesh = plsc.VectorSubcoreMesh(core_axis_name="c", subcore_axis_name="s")

    @functools.partial(
        pl.kernel,
        mesh=mesh,
        out_type=jax.ShapeDtypeStruct((N, D), jnp.float32),
        scratch_types=[
            pltpu.VMEM_SHARED((V, D), jnp.float32),
            pltpu.VMEM((B_MAIN,), jnp.int32),
            pltpu.VMEM((NBUF, BLK, D), jnp.float32),
            pltpu.SemaphoreType.DMA((NBUF,)),
            pltpu.SemaphoreType.DMA((NBUF,)),
            pltpu.SemaphoreType.DMA,
        ],
    )
    def k(table_hbm, idx_hbm, out_hbm, table_sh, idx_v, rows, sem_g, sem_w,
          sem_i):
        cid = lax.axis_index("c")
        sid = lax.axis_index("s")
        wid = sid * 2 + cid
        last = wid == NW - 1
        base = wid * B_MAIN

        @pl.when(jnp.logical_not(last))
        def _():
            pltpu.make_async_copy(
                idx_hbm.at[pl.ds(base, B_MAIN)], idx_v, sem_i).start()

        @pl.when(last)
        def _():
            pltpu.make_async_copy(
                idx_hbm.at[pl.ds(31 * B_MAIN, B_LAST)],
                idx_v.at[pl.ds(0, B_LAST)], sem_i).start()

        @pl.when(sid == 0)
        def _():
            pltpu.sync_copy(table_hbm, table_sh)

        plsc.subcore_barrier()

        @pl.when(jnp.logical_not(last))
        def _():
            pltpu.make_async_copy(
                idx_hbm.at[pl.ds(base, B_MAIN)], idx_v, sem_i).wait()

        @pl.when(last)
        def _():
            pltpu.make_async_copy(
                idx_hbm.at[pl.ds(31 * B_MAIN, B_LAST)],
                idx_v.at[pl.ds(0, B_LAST)], sem_i).wait()

        def g_copy(j, b):
            return pltpu.make_async_copy(
                table_sh.at[idx_v.at[pl.ds(j * BLK, BLK)]],
                rows.at[b],
                sem_g.at[b],
            )

        def w_copy(j, b):
            return pltpu.make_async_copy(
                rows.at[b],
                out_hbm.at[pl.ds(base + j * BLK, BLK)],
                sem_w.at[b],
            )

        for kk in range(LOOK):
            g_copy(kk, kk).start()

        def body(j, carry):
            b = j % NBUF
            g_copy(j, b).wait()
            w_copy(j, b).start()
            nxt = j + LOOK

            @pl.when(nxt < NFULL)
            def _():
                prev = nxt - NBUF

                @pl.when(prev >= 0)
                def _():
                    w_copy(prev, prev % NBUF).wait()

                g_copy(nxt, nxt % NBUF).start()

            return carry

        lax.fori_loop(0, NFULL, body, 0)

        w_copy(NFULL - NBUF, (NFULL - NBUF) % NBUF).wait()

        def tail(tsz):
            toff = NFULL * BLK
            b = NFULL % NBUF
            pltpu.make_async_copy(
                table_sh.at[idx_v.at[pl.ds(toff, tsz)]],
                rows.at[b, pl.ds(0, tsz)],
                sem_g.at[b],
            ).start()
            pltpu.make_async_copy(
                table_sh.at[idx_v.at[pl.ds(toff, tsz)]],
                rows.at[b, pl.ds(0, tsz)],
                sem_g.at[b],
            ).wait()
            pltpu.make_async_copy(
                rows.at[b, pl.ds(0, tsz)],
                out_hbm.at[pl.ds(base + toff, tsz)],
                sem_w.at[b],
            ).start()
            pltpu.make_async_copy(
                rows.at[b, pl.ds(0, tsz)],
                out_hbm.at[pl.ds(base + toff, tsz)],
                sem_w.at[b],
            ).wait()

        @pl.when(jnp.logical_not(last))
        def _():
            tail(T_MAIN)

        @pl.when(last)
        def _():
            tail(T_LAST)

        def drain(j, carry):
            w_copy(j, j % NBUF).wait()
            return carry

        lax.fori_loop(NFULL - NBUF + 1, NFULL, drain, 0)

    return k


_kernel = _make_kernel()


def kernel(atomic_num, embedding_table):
    idx = atomic_num.astype(jnp.int32)
    return _kernel(embedding_table, idx)


# speedup vs baseline: 4.0829x; 1.0007x over previous
import functools

import jax
import jax.numpy as jnp
from jax import lax
from jax.experimental import pallas as pl
from jax.experimental.pallas import tpu as pltpu
from jax.experimental.pallas import tpu_sc as plsc

N = 100000
D = 128
V = 95
NW = 32
B_MAIN = 3128
B_LAST = N - 31 * B_MAIN
BLK = 112
NBUF = 8
LOOK = 6
NFULL = 27
T_MAIN = B_MAIN - NFULL * BLK
T_LAST = B_LAST - NFULL * BLK


def _make_kernel():
    mesh = plsc.VectorSubcoreMesh(core_axis_name="c", subcore_axis_name="s")

    @functools.partial(
        pl.kernel,
        mesh=mesh,
        out_type=jax.ShapeDtypeStruct((N, D), jnp.float32),
        scratch_types=[
            pltpu.VMEM_SHARED((V, D), jnp.float32),
            pltpu.VMEM((B_MAIN,), jnp.int32),
            pltpu.VMEM((NBUF, BLK, D), jnp.float32),
            pltpu.SemaphoreType.DMA((NBUF,)),
            pltpu.SemaphoreType.DMA((NBUF,)),
            pltpu.SemaphoreType.DMA,
        ],
    )
    def k(table_hbm, idx_hbm, out_hbm, table_sh, idx_v, rows, sem_g, sem_w,
          sem_i):
        cid = lax.axis_index("c")
        sid = lax.axis_index("s")
        wid = sid * 2 + cid
        last = wid == NW - 1
        base = wid * B_MAIN

        @pl.when(jnp.logical_not(last))
        def _():
            pltpu.make_async_copy(
                idx_hbm.at[pl.ds(base, B_MAIN)], idx_v, sem_i).start()

        @pl.when(last)
        def _():
            pltpu.make_async_copy(
                idx_hbm.at[pl.ds(31 * B_MAIN, B_LAST)],
                idx_v.at[pl.ds(0, B_LAST)], sem_i).start()

        @pl.when(sid == 0)
        def _():
            pltpu.sync_copy(table_hbm, table_sh)

        plsc.subcore_barrier()

        @pl.when(jnp.logical_not(last))
        def _():
            pltpu.make_async_copy(
                idx_hbm.at[pl.ds(base, B_MAIN)], idx_v, sem_i).wait()

        @pl.when(last)
        def _():
            pltpu.make_async_copy(
                idx_hbm.at[pl.ds(31 * B_MAIN, B_LAST)],
                idx_v.at[pl.ds(0, B_LAST)], sem_i).wait()

        def g_copy(j, b):
            return pltpu.make_async_copy(
                table_sh.at[idx_v.at[pl.ds(j * BLK, BLK)]],
                rows.at[b],
                sem_g.at[b],
            )

        def w_copy(j, b):
            return pltpu.make_async_copy(
                rows.at[b],
                out_hbm.at[pl.ds(base + j * BLK, BLK)],
                sem_w.at[b],
            )

        for kk in range(LOOK):
            g_copy(kk, kk).start()

        def body(j, carry):
            b = j % NBUF
            g_copy(j, b).wait()
            w_copy(j, b).start()
            nxt = j + LOOK

            @pl.when(nxt < NFULL)
            def _():
                prev = nxt - NBUF

                @pl.when(prev >= 0)
                def _():
                    w_copy(prev, prev % NBUF).wait()

                g_copy(nxt, nxt % NBUF).start()

            return carry

        lax.fori_loop(0, NFULL, body, 0)

        w_copy(NFULL - NBUF, (NFULL - NBUF) % NBUF).wait()

        def tail(tsz):
            toff = NFULL * BLK
            b = NFULL % NBUF
            pltpu.make_async_copy(
                table_sh.at[idx_v.at[pl.ds(toff, tsz)]],
                rows.at[b, pl.ds(0, tsz)],
                sem_g.at[b],
            ).start()
            pltpu.make_async_copy(
                table_sh.at[idx_v.at[pl.ds(toff, tsz)]],
                rows.at[b, pl.ds(0, tsz)],
                sem_g.at[b],
            ).wait()
            pltpu.make_async_copy(
                rows.at[b, pl.ds(0, tsz)],
                out_hbm.at[pl.ds(base + toff, tsz)],
                sem_w.at[b],
            ).start()
            pltpu.make_async_copy(
                rows.at[b, pl.ds(0, tsz)],
                out_hbm.at[pl.ds(base + toff, tsz)],
                sem_w.at[b],
            ).wait()

        @pl.when(jnp.logical_not(last))
        def _():
            tail(T_MAIN)

        @pl.when(last)
        def _():
            tail(T_LAST)

        def drain(j, carry):
            w_copy(j, j % NBUF).wait()
            return carry

        lax.fori_loop(NFULL - NBUF + 1, NFULL, drain, 0)

    return k


_kernel = _make_kernel()


def kernel(atomic_num, embedding_table):
    idx = atomic_num.astype(jnp.int32)
    return _kernel(embedding_table, idx)
